# Initial kernel scaffold; baseline (speedup 1.0000x reference)
import jax
import jax.numpy as jnp
from jax.experimental import pallas as pl


def kernel(x, edge_index, W_s0, b_s0, W_o0, b_o0, W_s1, b_s1, W_o1, b_o1, W_s2, b_s2, W_o2, b_o2):
    raise NotImplementedError("write your pallas kernel here")


# speedup vs baseline: 3.6337x; 3.6337x over previous
import functools

import jax
import jax.numpy as jnp
from jax import lax
from jax.experimental import pallas as pl
from jax.experimental.pallas import tpu as pltpu
from jax.experimental.pallas import tpu_sc as plsc

N = 10000
E = 160000
S = 256
O = 128
L = 3
T = 3

NC = 2
NS = 16
CH = 128
EPAD = ((E + NS * CH - 1) // (NS * CH)) * (NS * CH)
NPAD = ((N + NS * 8 - 1) // (NS * 8)) * (NS * 8)


def _segsum_sc(s2n, srcs, dsts, zeros):
    mesh = plsc.VectorSubcoreMesh(core_axis_name="c", subcore_axis_name="s")
    per_tile = EPAD // NS
    n_chunks = per_tile // CH
    zrows = NPAD // NS
    wrows = NPAD // NS

    @functools.partial(
        pl.kernel,
        out_type=jax.ShapeDtypeStruct((NC, NPAD, 128), jnp.float32),
        mesh=mesh,
        scratch_types=[
            pltpu.VMEM((CH,), jnp.int32),
            pltpu.VMEM((CH,), jnp.int32),
            pltpu.VMEM((CH, 128), jnp.float32),
            pltpu.VMEM_SHARED((NPAD, 128), jnp.float32),
            pltpu.SemaphoreType.DMA,
        ],
    )
    def seg_kernel(s_hbm, src_hbm, dst_hbm, zero_hbm, out_hbm,
                   sidx, didx, rows, agg, sem):
        c = lax.axis_index("c")
        t = lax.axis_index("s")
        pltpu.sync_copy(zero_hbm.at[pl.ds(t * zrows, zrows)],
                        agg.at[pl.ds(t * zrows, zrows)])
        plsc.subcore_barrier()

        def body(g, carry):
            base = t * per_tile + g * CH
            pltpu.sync_copy(src_hbm.at[c, pl.ds(base, CH)], sidx)
            pltpu.sync_copy(dst_hbm.at[pl.ds(base, CH)], didx)
            pltpu.async_copy(s_hbm.at[sidx], rows, sem).wait()
            pltpu.sync_copy(rows, agg.at[didx], add=True)
            return carry

        lax.fori_loop(0, n_chunks, body, 0)
        plsc.subcore_barrier()
        pltpu.sync_copy(agg.at[pl.ds(t * wrows, wrows)],
                        out_hbm.at[c, pl.ds(t * wrows, wrows)])

    return seg_kernel(s2n, srcs, dsts, zeros)


def _tc_mm(a_ws, bias, c_arr, want_pre, want_act, want_stk, rb=1000):
    n = a_ws[0][0].shape[0]
    m = a_ws[0][1].shape[1]
    grid = (n // rb,)
    in_specs = []
    operands = []
    for (a, w) in a_ws:
        k = a.shape[1]
        in_specs.append(pl.BlockSpec((rb, k), lambda i: (i, 0)))
        in_specs.append(pl.BlockSpec((k, m), lambda i: (0, 0)))
        operands += [a, w]
    if bias is not None:
        in_specs.append(pl.BlockSpec((1, m), lambda i: (0, 0)))
        operands.append(bias.reshape(1, m))
    if c_arr is not None:
        in_specs.append(pl.BlockSpec((rb, m), lambda i: (i, 0)))
        operands.append(c_arr)

    out_shapes, out_specs = [], []
    if want_pre:
        out_shapes.append(jax.ShapeDtypeStruct((n, m), jnp.float32))
        out_specs.append(pl.BlockSpec((rb, m), lambda i: (i, 0)))
    if want_act:
        out_shapes.append(jax.ShapeDtypeStruct((n, m), jnp.float32))
        out_specs.append(pl.BlockSpec((rb, m), lambda i: (i, 0)))
    if want_stk:
        out_shapes.append(jax.ShapeDtypeStruct((2, n, m // 2), jnp.float32))
        out_specs.append(pl.BlockSpec((2, rb, m // 2), lambda i: (0, i, 0)))

    n_a = len(a_ws)
    has_b = bias is not None
    has_c = c_arr is not None

    def body(*refs):
        pos = 2 * n_a
        acc = None
        for j in range(n_a):
            prod = jnp.dot(refs[2 * j][...], refs[2 * j + 1][...],
                           preferred_element_type=jnp.float32)
            acc = prod if acc is None else acc + prod
        if has_b:
            acc = acc + refs[pos][...]
            pos += 1
        if has_c:
            acc = acc + refs[pos][...]
            pos += 1
        outs = refs[pos:]
        oi = 0
        if want_pre:
            outs[oi][...] = acc
            oi += 1
        if want_act or want_stk:
            act = jnp.tanh(acc)
        if want_act:
            outs[oi][...] = act
            oi += 1
        if want_stk:
            outs[oi][0] = act[:, : m // 2]
            outs[oi][1] = act[:, m // 2:]

    res = pl.pallas_call(
        body,
        grid=grid,
        in_specs=in_specs,
        out_specs=out_specs if len(out_specs) > 1 else out_specs[0],
        out_shape=out_shapes if len(out_shapes) > 1 else out_shapes[0],
    )(*operands)
    return res if isinstance(res, (list, tuple)) else (res,)


def kernel(x, edge_index, W_s0, b_s0, W_o0, b_o0, W_s1, b_s1, W_o1, b_o1,
           W_s2, b_s2, W_o2, b_o2):
    src = edge_index[0]
    dst = edge_index[1]
    src_pad = jnp.concatenate(
        [src, jnp.zeros((EPAD - E,), jnp.int32)])
    srcs = jnp.stack([src_pad, src_pad + N])
    dsts = jnp.concatenate(
        [dst, jnp.full((EPAD - E,), N, jnp.int32)])
    zeros = jnp.zeros((NPAD, 128), jnp.float32)

    params = [(W_s0, b_s0, W_o0, b_o0), (W_s1, b_s1, W_o1, b_o1),
              (W_s2, b_s2, W_o2, b_o2)]
    dims = [256, 640, 1024]
    h = x
    out = None
    for l in range(L):
        W_s, b_s, W_o, b_o = params[l]
        d = dims[l]
        W_h, W_a = W_s[:d], W_s[d:]
        hWb, s, s_stk = _tc_mm([(h, W_h)], b_s, None, True, True, True)
        for t in range(1, T):
            agg = _segsum_sc(s_stk.reshape(2 * N, 128), srcs, dsts, zeros)
            agg = agg[:, :N]
            need_stk = t < T - 1
            res = _tc_mm([(agg[0], W_a[:128]), (agg[1], W_a[128:])],
                         None, hWb, False, True, need_stk)
            s = res[0]
            if need_stk:
                s_stk = res[1]
        (out,) = _tc_mm([(h, W_o[:d]), (s, W_o[d:])], b_o, None,
                        True, False, False)
        if l < L - 1:
            h = jnp.concatenate([h, s, out], axis=1)
    return out
